# baseline (speedup 1.0000x reference)
import functools

import jax
import jax.numpy as jnp
from jax import lax
from jax.experimental import pallas as pl
from jax.experimental.pallas import tpu as pltpu
from jax.experimental.pallas import tpu_sc as plsc

N = 10000
E = 160000
D_IN = 256
D_H = 512

NPAD = 10240
EPAD = 163840
B = 128
NBLK = EPAD // (16 * B)
GRP = 16
ROWS_PER_TILE = NPAD // 16

_mesh = plsc.VectorSubcoreMesh(core_axis_name="c", subcore_axis_name="s")


def _sc_agg_body(nch_per_core, with_deg, table_hbm, src_hbm, dst_hbm,
                 zeros_hbm, ones_hbm, agg_out, deg_out,
                 sidx_s, didx_s, rowsA, rowsB, agg_sh,
                 semGA, semGB, semSA, semSB):
    cid = lax.axis_index("c")
    sid = lax.axis_index("s")
    row0 = sid * ROWS_PER_TILE

    def wait64k(sem):
        pltpu.make_async_copy(zeros_hbm, rowsA, sem).wait()

    def zero_table():
        pltpu.sync_copy(zeros_hbm, rowsA)
        for k in range(ROWS_PER_TILE // B):
            pltpu.sync_copy(rowsA, agg_sh.at[pl.ds(row0 + k * B, B)])

    def copy_out(dst_ref, base):
        bufs = (rowsA, rowsB)
        sems = (semSA, semSB)
        n = ROWS_PER_TILE // B
        for k in range(n):
            p = k % 2
            if k >= 2:
                pltpu.make_async_copy(zeros_hbm, bufs[p], sems[p]).wait()
            pltpu.sync_copy(agg_sh.at[pl.ds(row0 + k * B, B)], bufs[p])
            pltpu.async_copy(bufs[p], dst_ref.at[pl.ds(base + row0 + k * B, B)],
                             sems[p])
        pltpu.make_async_copy(zeros_hbm, bufs[(n - 1) % 2], sems[(n - 1) % 2]).wait()
        pltpu.make_async_copy(zeros_hbm, bufs[n % 2], sems[n % 2]).wait()

    for p in range(nch_per_core):
        chunk = cid * nch_per_core + p
        zero_table()
        plsc.subcore_barrier()

        @pl.loop(0, NBLK // GRP)
        def _(g):
            base = sid * NBLK + g * GRP
            pltpu.sync_copy(src_hbm.at[pl.ds(chunk * (EPAD // B) + base, GRP)],
                            sidx_s)
            pltpu.sync_copy(dst_hbm.at[pl.ds(base, GRP)], didx_s)
            pltpu.async_copy(table_hbm.at[sidx_s.at[0]], rowsA, semGA)
            pltpu.async_copy(table_hbm.at[sidx_s.at[1]], rowsB, semGB)

            @pl.loop(0, GRP // 2)
            def _(t):
                j0 = 2 * t
                pltpu.make_async_copy(zeros_hbm, rowsA, semGA).wait()
                pltpu.async_copy(rowsA, agg_sh.at[didx_s.at[j0]], semSA,
                                 add=True)
                pltpu.make_async_copy(zeros_hbm, rowsB, semGB).wait()
                pltpu.async_copy(rowsB, agg_sh.at[didx_s.at[j0 + 1]], semSB,
                                 add=True)
                pltpu.make_async_copy(zeros_hbm, rowsA, semSA).wait()
                pltpu.async_copy(
                    table_hbm.at[sidx_s.at[jnp.minimum(j0 + 2, GRP - 1)]],
                    rowsA, semGA)
                pltpu.make_async_copy(zeros_hbm, rowsB, semSB).wait()
                pltpu.async_copy(
                    table_hbm.at[sidx_s.at[jnp.minimum(j0 + 3, GRP - 1)]],
                    rowsB, semGB)

            pltpu.make_async_copy(zeros_hbm, rowsA, semGA).wait()
            pltpu.make_async_copy(zeros_hbm, rowsB, semGB).wait()

        plsc.subcore_barrier()
        copy_out(agg_out, chunk * NPAD)

    if with_deg:
        zero_table()
        pltpu.sync_copy(ones_hbm, rowsB)
        plsc.subcore_barrier()
        lo = cid * (NBLK // 2)

        @pl.loop(0, NBLK // 2 // 8)
        def _(g):
            base = sid * NBLK + lo + g * 8
            pltpu.sync_copy(dst_hbm.at[pl.ds(base, 8)], didx_s.at[pl.ds(0, 8)])
            for jj in range(8):
                pltpu.async_copy(rowsB, agg_sh.at[didx_s.at[jj]], semSA,
                                 add=True)
            for jj in range(8):
                pltpu.make_async_copy(zeros_hbm, rowsA, semSA).wait()

        plsc.subcore_barrier()
        copy_out(deg_out, cid * NPAD)


def _make_sc_agg1():
    scratch = [
        pltpu.VMEM((GRP, B), jnp.int32),
        pltpu.VMEM((GRP, B), jnp.int32),
        pltpu.VMEM((B, 128), jnp.float32),
        pltpu.VMEM((B, 128), jnp.float32),
        pltpu.VMEM_SHARED((NPAD, 128), jnp.float32),
        pltpu.SemaphoreType.DMA,
        pltpu.SemaphoreType.DMA,
        pltpu.SemaphoreType.DMA,
        pltpu.SemaphoreType.DMA,
    ]
    out_t = [jax.ShapeDtypeStruct((2 * NPAD, 128), jnp.float32),
             jax.ShapeDtypeStruct((2 * NPAD, 128), jnp.float32)]

    @functools.partial(pl.kernel, mesh=_mesh, out_type=out_t, scratch_types=scratch)
    def sc_agg1(table, src, dst, zeros, ones, agg_out, deg_out,
                sidx_s, didx_s, rowsA, rowsB, agg_sh, semGA, semGB, semSA,
                semSB):
        _sc_agg_body(1, True, table, src, dst, zeros, ones, agg_out, deg_out,
                     sidx_s, didx_s, rowsA, rowsB, agg_sh, semGA, semGB,
                     semSA, semSB)

    return sc_agg1


def _make_sc_agg2():
    scratch = [
        pltpu.VMEM((GRP, B), jnp.int32),
        pltpu.VMEM((GRP, B), jnp.int32),
        pltpu.VMEM((B, 128), jnp.float32),
        pltpu.VMEM((B, 128), jnp.float32),
        pltpu.VMEM_SHARED((NPAD, 128), jnp.float32),
        pltpu.SemaphoreType.DMA,
        pltpu.SemaphoreType.DMA,
        pltpu.SemaphoreType.DMA,
        pltpu.SemaphoreType.DMA,
    ]
    out_t = jax.ShapeDtypeStruct((4 * NPAD, 128), jnp.float32)

    @functools.partial(pl.kernel, mesh=_mesh, out_type=out_t, scratch_types=scratch)
    def sc_agg2(table, src, dst, zeros, agg_out, sidx_s, didx_s, rowsA, rowsB,
                agg_sh, semGA, semGB, semSA, semSB):
        _sc_agg_body(2, False, table, src, dst, zeros, None, agg_out, None,
                     sidx_s, didx_s, rowsA, rowsB, agg_sh, semGA, semGB,
                     semSA, semSB)

    return sc_agg2


_sc_agg1 = _make_sc_agg1()
_sc_agg2 = _make_sc_agg2()

RB = 1024


def _tc1_body(x_ref, agg_ref, degp_ref, ws_ref, wn_ref, b_ref, out_ref):
    deg = degp_ref[0][:, 0:1] + degp_ref[1][:, 0:1]
    invd = 1.0 / jnp.maximum(deg, 1.0)
    mean = jnp.concatenate([agg_ref[0], agg_ref[1]], axis=1) * invd
    acc = jnp.dot(x_ref[...], ws_ref[...], preferred_element_type=jnp.float32)
    acc = acc + jnp.dot(mean, wn_ref[...], preferred_element_type=jnp.float32)
    h = jnp.maximum(acc + b_ref[...], 0.0)
    for j in range(4):
        out_ref[j] = h[:, j * 128:(j + 1) * 128]


def _tc2_body(h_ref, agg_ref, degp_ref, ws_ref, wn_ref, b_ref, out_ref):
    deg = degp_ref[0][:, 0:1] + degp_ref[1][:, 0:1]
    invd = 1.0 / jnp.maximum(deg, 1.0)
    hb = jnp.concatenate([h_ref[j] for j in range(4)], axis=1)
    mean = jnp.concatenate([agg_ref[j] for j in range(4)], axis=1) * invd
    acc = jnp.dot(hb, ws_ref[...], preferred_element_type=jnp.float32)
    acc = acc + jnp.dot(mean, wn_ref[...], preferred_element_type=jnp.float32)
    out_ref[...] = acc + b_ref[...]


def _tc_layer1(xpad, agg1, degp, W1_self, W1_neigh, b1):
    grid = (NPAD // RB,)
    return pl.pallas_call(
        _tc1_body,
        grid=grid,
        in_specs=[
            pl.BlockSpec((RB, D_IN), lambda i: (i, 0)),
            pl.BlockSpec((2, RB, 128), lambda i: (0, i, 0)),
            pl.BlockSpec((2, RB, 128), lambda i: (0, i, 0)),
            pl.BlockSpec((D_IN, D_H), lambda i: (0, 0)),
            pl.BlockSpec((D_IN, D_H), lambda i: (0, 0)),
            pl.BlockSpec((1, D_H), lambda i: (0, 0)),
        ],
        out_specs=pl.BlockSpec((4, RB, 128), lambda i: (0, i, 0)),
        out_shape=jax.ShapeDtypeStruct((4, NPAD, 128), jnp.float32),
    )(xpad, agg1, degp, W1_self, W1_neigh, b1)


def _tc_layer2(h4, agg2, degp, W2_self, W2_neigh, b2):
    grid = (NPAD // RB,)
    return pl.pallas_call(
        _tc2_body,
        grid=grid,
        in_specs=[
            pl.BlockSpec((4, RB, 128), lambda i: (0, i, 0)),
            pl.BlockSpec((4, RB, 128), lambda i: (0, i, 0)),
            pl.BlockSpec((2, RB, 128), lambda i: (0, i, 0)),
            pl.BlockSpec((D_H, D_H), lambda i: (0, 0)),
            pl.BlockSpec((D_H, D_H), lambda i: (0, 0)),
            pl.BlockSpec((1, D_H), lambda i: (0, 0)),
        ],
        out_specs=pl.BlockSpec((RB, D_H), lambda i: (i, 0)),
        out_shape=jax.ShapeDtypeStruct((NPAD, D_H), jnp.float32),
    )(h4, agg2, degp, W2_self, W2_neigh, b2)


def kernel(in_feat, edge_index, W1_self, W1_neigh, b1, W2_self, W2_neigh, b2):
    src = edge_index[0].astype(jnp.int32)
    dst = edge_index[1].astype(jnp.int32)
    pad_e = EPAD - E
    srcp = jnp.concatenate([src, jnp.zeros((pad_e,), jnp.int32)])
    dstp = jnp.concatenate([dst, jnp.full((pad_e,), NPAD - 1, jnp.int32)])
    src1 = jnp.concatenate([srcp, srcp + NPAD]).reshape(2 * EPAD // B, B)
    src2 = jnp.concatenate([srcp + c * NPAD for c in range(4)]).reshape(4 * EPAD // B, B)
    dst2d = dstp.reshape(EPAD // B, B)

    xpad = jnp.pad(in_feat, ((0, NPAD - N), (0, 0)))
    x2 = xpad.reshape(NPAD, 2, 128).transpose(1, 0, 2).reshape(2 * NPAD, 128)
    zeros = jnp.zeros((B, 128), jnp.float32)
    ones = jnp.ones((B, 128), jnp.float32)

    agg1, degp = _sc_agg1(x2, src1, dst2d, zeros, ones)
    h4 = _tc_layer1(xpad, agg1.reshape(2, NPAD, 128), degp.reshape(2, NPAD, 128),
                    W1_self, W1_neigh, b1.reshape(1, D_H))
    agg2 = _sc_agg2(h4.reshape(4 * NPAD, 128), src2, dst2d, zeros)
    out = _tc_layer2(h4, agg2.reshape(4, NPAD, 128), degp.reshape(2, NPAD, 128),
                     W2_self, W2_neigh, b2.reshape(1, D_H))
    return out[:N]


# speedup vs baseline: 1.1066x; 1.1066x over previous
import functools

import jax
import jax.numpy as jnp
from jax import lax
from jax.experimental import pallas as pl
from jax.experimental.pallas import tpu as pltpu
from jax.experimental.pallas import tpu_sc as plsc

N = 10000
E = 160000
D_IN = 256
D_H = 512

NPAD = 10240
EPAD = 163840
B = 128
NBLK = EPAD // (16 * B)
GRP = 40
ROWS_PER_TILE = NPAD // 16

_mesh = plsc.VectorSubcoreMesh(core_axis_name="c", subcore_axis_name="s")


def _sc_agg_body(nch_per_core, with_deg, table_hbm, src_hbm, dst_hbm,
                 zeros_hbm, ones_hbm, agg_out, deg_out,
                 sidx_s, didx_s, rowsA, rowsB, agg_sh,
                 semGA, semGB, semSA, semSB):
    cid = lax.axis_index("c")
    sid = lax.axis_index("s")
    row0 = sid * ROWS_PER_TILE

    def wait64k(sem):
        pltpu.make_async_copy(zeros_hbm, rowsA, sem).wait()

    def zero_table():
        pltpu.sync_copy(zeros_hbm, rowsA)
        for k in range(ROWS_PER_TILE // B):
            pltpu.sync_copy(rowsA, agg_sh.at[pl.ds(row0 + k * B, B)])

    def copy_out(dst_ref, base):
        bufs = (rowsA, rowsB)
        sems = (semSA, semSB)
        n = ROWS_PER_TILE // B
        for k in range(n):
            p = k % 2
            if k >= 2:
                pltpu.make_async_copy(zeros_hbm, bufs[p], sems[p]).wait()
            pltpu.sync_copy(agg_sh.at[pl.ds(row0 + k * B, B)], bufs[p])
            pltpu.async_copy(bufs[p], dst_ref.at[pl.ds(base + row0 + k * B, B)],
                             sems[p])
        pltpu.make_async_copy(zeros_hbm, bufs[(n - 1) % 2], sems[(n - 1) % 2]).wait()
        pltpu.make_async_copy(zeros_hbm, bufs[n % 2], sems[n % 2]).wait()

    for p in range(nch_per_core):
        chunk = cid * nch_per_core + p
        zero_table()
        plsc.subcore_barrier()

        @pl.loop(0, NBLK // GRP)
        def _(g):
            base = sid * NBLK + g * GRP
            pltpu.sync_copy(src_hbm.at[pl.ds(chunk * (EPAD // B) + base, GRP)],
                            sidx_s)
            pltpu.sync_copy(dst_hbm.at[pl.ds(base, GRP)], didx_s)
            pltpu.async_copy(table_hbm.at[sidx_s.at[0]], rowsA, semGA)

            @pl.loop(0, GRP // 2)
            def _(t):
                j0 = 2 * t
                pltpu.make_async_copy(zeros_hbm, rowsA, semGA).wait()
                pltpu.async_copy(table_hbm.at[sidx_s.at[j0 + 1]], rowsB, semGB)
                pltpu.sync_copy(rowsA, agg_sh.at[didx_s.at[j0]], add=True)
                pltpu.make_async_copy(zeros_hbm, rowsB, semGB).wait()
                pltpu.async_copy(
                    table_hbm.at[sidx_s.at[jnp.minimum(j0 + 2, GRP - 1)]],
                    rowsA, semGA)
                pltpu.sync_copy(rowsB, agg_sh.at[didx_s.at[j0 + 1]], add=True)

            pltpu.make_async_copy(zeros_hbm, rowsA, semGA).wait()

        plsc.subcore_barrier()
        copy_out(agg_out, chunk * NPAD)

    if with_deg:
        zero_table()
        pltpu.sync_copy(ones_hbm, rowsB)
        plsc.subcore_barrier()
        lo = cid * (NBLK // 2)

        @pl.loop(0, NBLK // 2 // 8)
        def _(g):
            base = sid * NBLK + lo + g * 8
            pltpu.sync_copy(dst_hbm.at[pl.ds(base, 8)], didx_s.at[pl.ds(0, 8)])
            for jj in range(8):
                pltpu.async_copy(rowsB, agg_sh.at[didx_s.at[jj]], semSA,
                                 add=True)
            for jj in range(8):
                pltpu.make_async_copy(zeros_hbm, rowsA, semSA).wait()

        plsc.subcore_barrier()
        copy_out(deg_out, cid * NPAD)


def _make_sc_agg1():
    scratch = [
        pltpu.VMEM((GRP, B), jnp.int32),
        pltpu.VMEM((GRP, B), jnp.int32),
        pltpu.VMEM((B, 128), jnp.float32),
        pltpu.VMEM((B, 128), jnp.float32),
        pltpu.VMEM_SHARED((NPAD, 128), jnp.float32),
        pltpu.SemaphoreType.DMA,
        pltpu.SemaphoreType.DMA,
        pltpu.SemaphoreType.DMA,
        pltpu.SemaphoreType.DMA,
    ]
    out_t = [jax.ShapeDtypeStruct((2 * NPAD, 128), jnp.float32),
             jax.ShapeDtypeStruct((2 * NPAD, 128), jnp.float32)]

    @functools.partial(pl.kernel, mesh=_mesh, out_type=out_t, scratch_types=scratch)
    def sc_agg1(table, src, dst, zeros, ones, agg_out, deg_out,
                sidx_s, didx_s, rowsA, rowsB, agg_sh, semGA, semGB, semSA,
                semSB):
        _sc_agg_body(1, True, table, src, dst, zeros, ones, agg_out, deg_out,
                     sidx_s, didx_s, rowsA, rowsB, agg_sh, semGA, semGB,
                     semSA, semSB)

    return sc_agg1


def _make_sc_agg2():
    scratch = [
        pltpu.VMEM((GRP, B), jnp.int32),
        pltpu.VMEM((GRP, B), jnp.int32),
        pltpu.VMEM((B, 128), jnp.float32),
        pltpu.VMEM((B, 128), jnp.float32),
        pltpu.VMEM_SHARED((NPAD, 128), jnp.float32),
        pltpu.SemaphoreType.DMA,
        pltpu.SemaphoreType.DMA,
        pltpu.SemaphoreType.DMA,
        pltpu.SemaphoreType.DMA,
    ]
    out_t = jax.ShapeDtypeStruct((4 * NPAD, 128), jnp.float32)

    @functools.partial(pl.kernel, mesh=_mesh, out_type=out_t, scratch_types=scratch)
    def sc_agg2(table, src, dst, zeros, agg_out, sidx_s, didx_s, rowsA, rowsB,
                agg_sh, semGA, semGB, semSA, semSB):
        _sc_agg_body(2, False, table, src, dst, zeros, None, agg_out, None,
                     sidx_s, didx_s, rowsA, rowsB, agg_sh, semGA, semGB,
                     semSA, semSB)

    return sc_agg2


_sc_agg1 = _make_sc_agg1()
_sc_agg2 = _make_sc_agg2()

RB = 1024


def _tc1_body(x_ref, agg_ref, degp_ref, ws_ref, wn_ref, b_ref, out_ref):
    deg = degp_ref[0][:, 0:1] + degp_ref[1][:, 0:1]
    invd = 1.0 / jnp.maximum(deg, 1.0)
    mean = jnp.concatenate([agg_ref[0], agg_ref[1]], axis=1) * invd
    acc = jnp.dot(x_ref[...], ws_ref[...], preferred_element_type=jnp.float32)
    acc = acc + jnp.dot(mean, wn_ref[...], preferred_element_type=jnp.float32)
    h = jnp.maximum(acc + b_ref[...], 0.0)
    for j in range(4):
        out_ref[j] = h[:, j * 128:(j + 1) * 128]


def _tc2_body(h_ref, agg_ref, degp_ref, ws_ref, wn_ref, b_ref, out_ref):
    deg = degp_ref[0][:, 0:1] + degp_ref[1][:, 0:1]
    invd = 1.0 / jnp.maximum(deg, 1.0)
    hb = jnp.concatenate([h_ref[j] for j in range(4)], axis=1)
    mean = jnp.concatenate([agg_ref[j] for j in range(4)], axis=1) * invd
    acc = jnp.dot(hb, ws_ref[...], preferred_element_type=jnp.float32)
    acc = acc + jnp.dot(mean, wn_ref[...], preferred_element_type=jnp.float32)
    out_ref[...] = acc + b_ref[...]


def _tc_layer1(xpad, agg1, degp, W1_self, W1_neigh, b1):
    grid = (NPAD // RB,)
    return pl.pallas_call(
        _tc1_body,
        grid=grid,
        in_specs=[
            pl.BlockSpec((RB, D_IN), lambda i: (i, 0)),
            pl.BlockSpec((2, RB, 128), lambda i: (0, i, 0)),
            pl.BlockSpec((2, RB, 128), lambda i: (0, i, 0)),
            pl.BlockSpec((D_IN, D_H), lambda i: (0, 0)),
            pl.BlockSpec((D_IN, D_H), lambda i: (0, 0)),
            pl.BlockSpec((1, D_H), lambda i: (0, 0)),
        ],
        out_specs=pl.BlockSpec((4, RB, 128), lambda i: (0, i, 0)),
        out_shape=jax.ShapeDtypeStruct((4, NPAD, 128), jnp.float32),
    )(xpad, agg1, degp, W1_self, W1_neigh, b1)


def _tc_layer2(h4, agg2, degp, W2_self, W2_neigh, b2):
    grid = (NPAD // RB,)
    return pl.pallas_call(
        _tc2_body,
        grid=grid,
        in_specs=[
            pl.BlockSpec((4, RB, 128), lambda i: (0, i, 0)),
            pl.BlockSpec((4, RB, 128), lambda i: (0, i, 0)),
            pl.BlockSpec((2, RB, 128), lambda i: (0, i, 0)),
            pl.BlockSpec((D_H, D_H), lambda i: (0, 0)),
            pl.BlockSpec((D_H, D_H), lambda i: (0, 0)),
            pl.BlockSpec((1, D_H), lambda i: (0, 0)),
        ],
        out_specs=pl.BlockSpec((RB, D_H), lambda i: (i, 0)),
        out_shape=jax.ShapeDtypeStruct((NPAD, D_H), jnp.float32),
    )(h4, agg2, degp, W2_self, W2_neigh, b2)


def kernel(in_feat, edge_index, W1_self, W1_neigh, b1, W2_self, W2_neigh, b2):
    src = edge_index[0].astype(jnp.int32)
    dst = edge_index[1].astype(jnp.int32)
    pad_e = EPAD - E
    srcp = jnp.concatenate([src, jnp.zeros((pad_e,), jnp.int32)])
    dstp = jnp.concatenate([dst, jnp.full((pad_e,), NPAD - 1, jnp.int32)])
    src1 = jnp.concatenate([srcp, srcp + NPAD]).reshape(2 * EPAD // B, B)
    src2 = jnp.concatenate([srcp + c * NPAD for c in range(4)]).reshape(4 * EPAD // B, B)
    dst2d = dstp.reshape(EPAD // B, B)

    xpad = jnp.pad(in_feat, ((0, NPAD - N), (0, 0)))
    x2 = xpad.reshape(NPAD, 2, 128).transpose(1, 0, 2).reshape(2 * NPAD, 128)
    zeros = jnp.zeros((B, 128), jnp.float32)
    ones = jnp.ones((B, 128), jnp.float32)

    agg1, degp = _sc_agg1(x2, src1, dst2d, zeros, ones)
    h4 = _tc_layer1(xpad, agg1.reshape(2, NPAD, 128), degp.reshape(2, NPAD, 128),
                    W1_self, W1_neigh, b1.reshape(1, D_H))
    agg2 = _sc_agg2(h4.reshape(4 * NPAD, 128), src2, dst2d, zeros)
    out = _tc_layer2(h4, agg2.reshape(4, NPAD, 128), degp.reshape(2, NPAD, 128),
                     W2_self, W2_neigh, b2.reshape(1, D_H))
    return out[:N]


# speedup vs baseline: 1.1779x; 1.0644x over previous
import functools

import jax
import jax.numpy as jnp
from jax import lax
from jax.experimental import pallas as pl
from jax.experimental.pallas import tpu as pltpu
from jax.experimental.pallas import tpu_sc as plsc

N = 10000
E = 160000
D_IN = 256
D_H = 512

NPAD = 10240
EPAD = 163840
B = 128
NBLK = EPAD // (16 * B)
GRP = 40
ROWS_PER_TILE = NPAD // 16

_mesh = plsc.VectorSubcoreMesh(core_axis_name="c", subcore_axis_name="s")


def _sc_agg_body(nch_per_core, with_deg, table_hbm, src_hbm, dst_hbm,
                 zeros_hbm, ones_hbm, agg_out, deg_out,
                 sidx_s, didx_s, rowsA, rowsB, agg_sh,
                 semGA, semGB, semSA, semSB):
    cid = lax.axis_index("c")
    sid = lax.axis_index("s")
    row0 = sid * ROWS_PER_TILE

    def wait64k(sem):
        pltpu.make_async_copy(zeros_hbm, rowsA, sem).wait()

    def zero_table():
        pltpu.sync_copy(zeros_hbm, rowsA)
        for k in range(ROWS_PER_TILE // B):
            pltpu.sync_copy(rowsA, agg_sh.at[pl.ds(row0 + k * B, B)])

    def copy_out(dst_ref, base):
        bufs = (rowsA, rowsB)
        sems = (semSA, semSB)
        n = ROWS_PER_TILE // B
        for k in range(n):
            p = k % 2
            if k >= 2:
                pltpu.make_async_copy(zeros_hbm, bufs[p], sems[p]).wait()
            pltpu.sync_copy(agg_sh.at[pl.ds(row0 + k * B, B)], bufs[p])
            pltpu.async_copy(bufs[p], dst_ref.at[pl.ds(base + row0 + k * B, B)],
                             sems[p])
        pltpu.make_async_copy(zeros_hbm, bufs[(n - 1) % 2], sems[(n - 1) % 2]).wait()
        pltpu.make_async_copy(zeros_hbm, bufs[n % 2], sems[n % 2]).wait()

    for p in range(nch_per_core):
        chunk = cid * nch_per_core + p
        zero_table()
        plsc.subcore_barrier()

        @pl.loop(0, NBLK // GRP)
        def _(g):
            base = sid * NBLK + g * GRP
            pltpu.sync_copy(src_hbm.at[pl.ds(chunk * (EPAD // B) + base, GRP)],
                            sidx_s)
            pltpu.sync_copy(dst_hbm.at[pl.ds(base, GRP)], didx_s)
            pltpu.async_copy(table_hbm.at[sidx_s.at[0]], rowsA, semGA)

            pltpu.async_copy(table_hbm.at[sidx_s.at[1]], rowsB, semGB)

            @pl.loop(0, GRP // 2)
            def _(t):
                j0 = 2 * t
                pltpu.make_async_copy(zeros_hbm, rowsA, semGA).wait()
                pltpu.sync_copy(rowsA, agg_sh.at[didx_s.at[j0]], add=True)
                pltpu.async_copy(
                    table_hbm.at[sidx_s.at[jnp.minimum(j0 + 2, GRP - 1)]],
                    rowsA, semGA)
                pltpu.make_async_copy(zeros_hbm, rowsB, semGB).wait()
                pltpu.sync_copy(rowsB, agg_sh.at[didx_s.at[j0 + 1]], add=True)
                pltpu.async_copy(
                    table_hbm.at[sidx_s.at[jnp.minimum(j0 + 3, GRP - 1)]],
                    rowsB, semGB)

            pltpu.make_async_copy(zeros_hbm, rowsA, semGA).wait()
            pltpu.make_async_copy(zeros_hbm, rowsB, semGB).wait()

        plsc.subcore_barrier()
        copy_out(agg_out, chunk * NPAD)

    if with_deg:
        zero_table()
        pltpu.sync_copy(ones_hbm, rowsB)
        plsc.subcore_barrier()
        lo = cid * (NBLK // 2)

        @pl.loop(0, NBLK // 2 // 8)
        def _(g):
            base = sid * NBLK + lo + g * 8
            pltpu.sync_copy(dst_hbm.at[pl.ds(base, 8)], didx_s.at[pl.ds(0, 8)])
            for jj in range(8):
                pltpu.async_copy(rowsB, agg_sh.at[didx_s.at[jj]], semSA,
                                 add=True)
            for jj in range(8):
                pltpu.make_async_copy(zeros_hbm, rowsA, semSA).wait()

        plsc.subcore_barrier()
        copy_out(deg_out, cid * NPAD)


def _make_sc_agg1():
    scratch = [
        pltpu.VMEM((GRP, B), jnp.int32),
        pltpu.VMEM((GRP, B), jnp.int32),
        pltpu.VMEM((B, 128), jnp.float32),
        pltpu.VMEM((B, 128), jnp.float32),
        pltpu.VMEM_SHARED((NPAD, 128), jnp.float32),
        pltpu.SemaphoreType.DMA,
        pltpu.SemaphoreType.DMA,
        pltpu.SemaphoreType.DMA,
        pltpu.SemaphoreType.DMA,
    ]
    out_t = [jax.ShapeDtypeStruct((2 * NPAD, 128), jnp.float32),
             jax.ShapeDtypeStruct((2 * NPAD, 128), jnp.float32)]

    @functools.partial(pl.kernel, mesh=_mesh, out_type=out_t, scratch_types=scratch)
    def sc_agg1(table, src, dst, zeros, ones, agg_out, deg_out,
                sidx_s, didx_s, rowsA, rowsB, agg_sh, semGA, semGB, semSA,
                semSB):
        _sc_agg_body(1, True, table, src, dst, zeros, ones, agg_out, deg_out,
                     sidx_s, didx_s, rowsA, rowsB, agg_sh, semGA, semGB,
                     semSA, semSB)

    return sc_agg1


def _make_sc_agg2():
    scratch = [
        pltpu.VMEM((GRP, B), jnp.int32),
        pltpu.VMEM((GRP, B), jnp.int32),
        pltpu.VMEM((B, 128), jnp.float32),
        pltpu.VMEM((B, 128), jnp.float32),
        pltpu.VMEM_SHARED((NPAD, 128), jnp.float32),
        pltpu.SemaphoreType.DMA,
        pltpu.SemaphoreType.DMA,
        pltpu.SemaphoreType.DMA,
        pltpu.SemaphoreType.DMA,
    ]
    out_t = jax.ShapeDtypeStruct((4 * NPAD, 128), jnp.float32)

    @functools.partial(pl.kernel, mesh=_mesh, out_type=out_t, scratch_types=scratch)
    def sc_agg2(table, src, dst, zeros, agg_out, sidx_s, didx_s, rowsA, rowsB,
                agg_sh, semGA, semGB, semSA, semSB):
        _sc_agg_body(2, False, table, src, dst, zeros, None, agg_out, None,
                     sidx_s, didx_s, rowsA, rowsB, agg_sh, semGA, semGB,
                     semSA, semSB)

    return sc_agg2


_sc_agg1 = _make_sc_agg1()
_sc_agg2 = _make_sc_agg2()

RB = 1024


def _self1_body(x_ref, w_ref, b_ref, out_ref):
    out_ref[...] = jnp.dot(x_ref[...], w_ref[...],
                           preferred_element_type=jnp.float32) + b_ref[...]


def _rest1_body(z_ref, agg_ref, degp_ref, wn_ref, out_ref):
    deg = degp_ref[0][:, 0:1] + degp_ref[1][:, 0:1]
    invd = 1.0 / jnp.maximum(deg, 1.0)
    mean = jnp.concatenate([agg_ref[0], agg_ref[1]], axis=1) * invd
    h = jnp.maximum(
        z_ref[...] + jnp.dot(mean, wn_ref[...],
                             preferred_element_type=jnp.float32), 0.0)
    for j in range(4):
        out_ref[j] = h[:, j * 128:(j + 1) * 128]


def _self2_body(h_ref, w_ref, b_ref, out_ref):
    hb = jnp.concatenate([h_ref[j] for j in range(4)], axis=1)
    out_ref[...] = jnp.dot(hb, w_ref[...],
                           preferred_element_type=jnp.float32) + b_ref[...]


def _rest2_body(z_ref, agg_ref, degp_ref, wn_ref, out_ref):
    deg = degp_ref[0][:, 0:1] + degp_ref[1][:, 0:1]
    invd = 1.0 / jnp.maximum(deg, 1.0)
    mean = jnp.concatenate([agg_ref[j] for j in range(4)], axis=1) * invd
    out_ref[...] = z_ref[...] + jnp.dot(mean, wn_ref[...],
                                        preferred_element_type=jnp.float32)


def _row_spec(w):
    return pl.BlockSpec((RB, w), lambda i: (i, 0))


def _chunk_spec(c):
    return pl.BlockSpec((c, RB, 128), lambda i: (0, i, 0))


def _full_spec(r, c):
    return pl.BlockSpec((r, c), lambda i: (0, 0))


def _tc_self1(xpad, W1_self, b1):
    return pl.pallas_call(
        _self1_body, grid=(NPAD // RB,),
        in_specs=[_row_spec(D_IN), _full_spec(D_IN, D_H), _full_spec(1, D_H)],
        out_specs=_row_spec(D_H),
        out_shape=jax.ShapeDtypeStruct((NPAD, D_H), jnp.float32),
    )(xpad, W1_self, b1)


def _tc_rest1(z1, agg1, degp, W1_neigh):
    return pl.pallas_call(
        _rest1_body, grid=(NPAD // RB,),
        in_specs=[_row_spec(D_H), _chunk_spec(2), _chunk_spec(2),
                  _full_spec(D_IN, D_H)],
        out_specs=_chunk_spec(4),
        out_shape=jax.ShapeDtypeStruct((4, NPAD, 128), jnp.float32),
    )(z1, agg1, degp, W1_neigh)


def _tc_self2(h4, W2_self, b2):
    return pl.pallas_call(
        _self2_body, grid=(NPAD // RB,),
        in_specs=[_chunk_spec(4), _full_spec(D_H, D_H), _full_spec(1, D_H)],
        out_specs=_row_spec(D_H),
        out_shape=jax.ShapeDtypeStruct((NPAD, D_H), jnp.float32),
    )(h4, W2_self, b2)


def _tc_rest2(z2, agg2, degp, W2_neigh):
    return pl.pallas_call(
        _rest2_body, grid=(NPAD // RB,),
        in_specs=[_row_spec(D_H), _chunk_spec(4), _chunk_spec(2),
                  _full_spec(D_H, D_H)],
        out_specs=_row_spec(D_H),
        out_shape=jax.ShapeDtypeStruct((NPAD, D_H), jnp.float32),
    )(z2, agg2, degp, W2_neigh)


def kernel(in_feat, edge_index, W1_self, W1_neigh, b1, W2_self, W2_neigh, b2):
    src = edge_index[0].astype(jnp.int32)
    dst = edge_index[1].astype(jnp.int32)
    pad_e = EPAD - E
    srcp = jnp.concatenate([src, jnp.zeros((pad_e,), jnp.int32)])
    dstp = jnp.concatenate([dst, jnp.full((pad_e,), NPAD - 1, jnp.int32)])
    src1 = jnp.concatenate([srcp, srcp + NPAD]).reshape(2 * EPAD // B, B)
    src2 = jnp.concatenate([srcp + c * NPAD for c in range(4)]).reshape(4 * EPAD // B, B)
    dst2d = dstp.reshape(EPAD // B, B)

    xpad = jnp.pad(in_feat, ((0, NPAD - N), (0, 0)))
    x2 = xpad.reshape(NPAD, 2, 128).transpose(1, 0, 2).reshape(2 * NPAD, 128)
    zeros = jnp.zeros((B, 128), jnp.float32)
    ones = jnp.ones((B, 128), jnp.float32)

    agg1, degp = _sc_agg1(x2, src1, dst2d, zeros, ones)
    z1 = _tc_self1(xpad, W1_self, b1.reshape(1, D_H))
    h4 = _tc_rest1(z1, agg1.reshape(2, NPAD, 128), degp.reshape(2, NPAD, 128),
                   W1_neigh)
    agg2 = _sc_agg2(h4.reshape(4 * NPAD, 128), src2, dst2d, zeros)
    z2 = _tc_self2(h4, W2_self, b2.reshape(1, D_H))
    out = _tc_rest2(z2, agg2.reshape(4, NPAD, 128), degp.reshape(2, NPAD, 128),
                    W2_neigh)
    return out[:N]


# speedup vs baseline: 1.1965x; 1.0158x over previous
import functools

import jax
import jax.numpy as jnp
from jax import lax
from jax.experimental import pallas as pl
from jax.experimental.pallas import tpu as pltpu
from jax.experimental.pallas import tpu_sc as plsc

N = 10000
E = 160000
D_IN = 256
D_H = 512

NPAD = 10240
EPAD = 163840
B = 128
NBLK = EPAD // (16 * B)
GRP = 40
ROWS_PER_TILE = NPAD // 16

_mesh = plsc.VectorSubcoreMesh(core_axis_name="c", subcore_axis_name="s")


def _sc_agg_body(nch_per_core, with_deg, table_hbm, src_hbm, dst_hbm,
                 zeros_hbm, ones_hbm, agg_out, deg_out,
                 sidx_s, didx_s, rowsA, rowsB, agg_sh,
                 semGA, semGB, semSA, semSB):
    cid = lax.axis_index("c")
    sid = lax.axis_index("s")
    row0 = sid * ROWS_PER_TILE

    def wait64k(sem):
        pltpu.make_async_copy(zeros_hbm, rowsA, sem).wait()

    def zero_table():
        pltpu.sync_copy(zeros_hbm, rowsA)
        for k in range(ROWS_PER_TILE // B):
            pltpu.async_copy(rowsA, agg_sh.at[pl.ds(row0 + k * B, B)], semSA)
        for k in range(ROWS_PER_TILE // B):
            pltpu.make_async_copy(zeros_hbm, rowsA, semSA).wait()

    def copy_out(dst_ref, base):
        bufs = (rowsA, rowsB)
        sems = (semSA, semSB)
        n = ROWS_PER_TILE // B
        for k in range(n):
            p = k % 2
            if k >= 2:
                pltpu.make_async_copy(zeros_hbm, bufs[p], sems[p]).wait()
            pltpu.sync_copy(agg_sh.at[pl.ds(row0 + k * B, B)], bufs[p])
            pltpu.async_copy(bufs[p], dst_ref.at[pl.ds(base + row0 + k * B, B)],
                             sems[p])
        pltpu.make_async_copy(zeros_hbm, bufs[(n - 1) % 2], sems[(n - 1) % 2]).wait()
        pltpu.make_async_copy(zeros_hbm, bufs[n % 2], sems[n % 2]).wait()

    for p in range(nch_per_core):
        chunk = cid * nch_per_core + p
        zero_table()
        plsc.subcore_barrier()

        @pl.loop(0, NBLK // GRP)
        def _(g):
            base = sid * NBLK + g * GRP
            pltpu.sync_copy(src_hbm.at[pl.ds(chunk * (EPAD // B) + base, GRP)],
                            sidx_s)
            pltpu.sync_copy(dst_hbm.at[pl.ds(base, GRP)], didx_s)
            pltpu.async_copy(table_hbm.at[sidx_s.at[0]], rowsA, semGA)

            pltpu.async_copy(table_hbm.at[sidx_s.at[1]], rowsB, semGB)

            @pl.loop(0, GRP // 2)
            def _(t):
                j0 = 2 * t
                pltpu.make_async_copy(zeros_hbm, rowsA, semGA).wait()
                pltpu.sync_copy(rowsA, agg_sh.at[didx_s.at[j0]], add=True)
                pltpu.async_copy(
                    table_hbm.at[sidx_s.at[jnp.minimum(j0 + 2, GRP - 1)]],
                    rowsA, semGA)
                pltpu.make_async_copy(zeros_hbm, rowsB, semGB).wait()
                pltpu.sync_copy(rowsB, agg_sh.at[didx_s.at[j0 + 1]], add=True)
                pltpu.async_copy(
                    table_hbm.at[sidx_s.at[jnp.minimum(j0 + 3, GRP - 1)]],
                    rowsB, semGB)

            pltpu.make_async_copy(zeros_hbm, rowsA, semGA).wait()
            pltpu.make_async_copy(zeros_hbm, rowsB, semGB).wait()

        plsc.subcore_barrier()
        copy_out(agg_out, chunk * NPAD)

    if with_deg:
        zero_table()
        pltpu.sync_copy(ones_hbm, rowsB)
        plsc.subcore_barrier()
        lo = cid * (NBLK // 2)

        pltpu.sync_copy(dst_hbm.at[pl.ds(sid * NBLK + lo, NBLK // 2)],
                        didx_s.at[pl.ds(0, NBLK // 2)])
        for g in range(NBLK // 2 // 8):
            for jj in range(8):
                pltpu.async_copy(rowsB, agg_sh.at[didx_s.at[g * 8 + jj]],
                                 semSA, add=True)
            for jj in range(8):
                pltpu.make_async_copy(zeros_hbm, rowsA, semSA).wait()

        plsc.subcore_barrier()
        copy_out(deg_out, cid * NPAD)


def _make_sc_agg1():
    scratch = [
        pltpu.VMEM((GRP, B), jnp.int32),
        pltpu.VMEM((GRP, B), jnp.int32),
        pltpu.VMEM((B, 128), jnp.float32),
        pltpu.VMEM((B, 128), jnp.float32),
        pltpu.VMEM_SHARED((NPAD, 128), jnp.float32),
        pltpu.SemaphoreType.DMA,
        pltpu.SemaphoreType.DMA,
        pltpu.SemaphoreType.DMA,
        pltpu.SemaphoreType.DMA,
    ]
    out_t = [jax.ShapeDtypeStruct((2 * NPAD, 128), jnp.float32),
             jax.ShapeDtypeStruct((2 * NPAD, 128), jnp.float32)]

    @functools.partial(pl.kernel, mesh=_mesh, out_type=out_t, scratch_types=scratch)
    def sc_agg1(table, src, dst, zeros, ones, agg_out, deg_out,
                sidx_s, didx_s, rowsA, rowsB, agg_sh, semGA, semGB, semSA,
                semSB):
        _sc_agg_body(1, True, table, src, dst, zeros, ones, agg_out, deg_out,
                     sidx_s, didx_s, rowsA, rowsB, agg_sh, semGA, semGB,
                     semSA, semSB)

    return sc_agg1


def _make_sc_agg2():
    scratch = [
        pltpu.VMEM((GRP, B), jnp.int32),
        pltpu.VMEM((GRP, B), jnp.int32),
        pltpu.VMEM((B, 128), jnp.float32),
        pltpu.VMEM((B, 128), jnp.float32),
        pltpu.VMEM_SHARED((NPAD, 128), jnp.float32),
        pltpu.SemaphoreType.DMA,
        pltpu.SemaphoreType.DMA,
        pltpu.SemaphoreType.DMA,
        pltpu.SemaphoreType.DMA,
    ]
    out_t = jax.ShapeDtypeStruct((4 * NPAD, 128), jnp.float32)

    @functools.partial(pl.kernel, mesh=_mesh, out_type=out_t, scratch_types=scratch)
    def sc_agg2(table, src, dst, zeros, agg_out, sidx_s, didx_s, rowsA, rowsB,
                agg_sh, semGA, semGB, semSA, semSB):
        _sc_agg_body(2, False, table, src, dst, zeros, None, agg_out, None,
                     sidx_s, didx_s, rowsA, rowsB, agg_sh, semGA, semGB,
                     semSA, semSB)

    return sc_agg2


_sc_agg1 = _make_sc_agg1()
_sc_agg2 = _make_sc_agg2()

RB = 1024


def _self1_body(x_ref, w_ref, b_ref, out_ref):
    out_ref[...] = jnp.dot(x_ref[...], w_ref[...],
                           preferred_element_type=jnp.float32) + b_ref[...]


def _rest1_body(z_ref, agg_ref, degp_ref, wn_ref, out_ref):
    deg = degp_ref[0][:, 0:1] + degp_ref[1][:, 0:1]
    invd = 1.0 / jnp.maximum(deg, 1.0)
    mean = jnp.concatenate([agg_ref[0], agg_ref[1]], axis=1) * invd
    h = jnp.maximum(
        z_ref[...] + jnp.dot(mean, wn_ref[...],
                             preferred_element_type=jnp.float32), 0.0)
    for j in range(4):
        out_ref[j] = h[:, j * 128:(j + 1) * 128]


def _self2_body(h_ref, w_ref, b_ref, out_ref):
    hb = jnp.concatenate([h_ref[j] for j in range(4)], axis=1)
    out_ref[...] = jnp.dot(hb, w_ref[...],
                           preferred_element_type=jnp.float32) + b_ref[...]


def _rest2_body(z_ref, agg_ref, degp_ref, wn_ref, out_ref):
    deg = degp_ref[0][:, 0:1] + degp_ref[1][:, 0:1]
    invd = 1.0 / jnp.maximum(deg, 1.0)
    mean = jnp.concatenate([agg_ref[j] for j in range(4)], axis=1) * invd
    out_ref[...] = z_ref[...] + jnp.dot(mean, wn_ref[...],
                                        preferred_element_type=jnp.float32)


def _row_spec(w):
    return pl.BlockSpec((RB, w), lambda i: (i, 0))


def _chunk_spec(c):
    return pl.BlockSpec((c, RB, 128), lambda i: (0, i, 0))


def _full_spec(r, c):
    return pl.BlockSpec((r, c), lambda i: (0, 0))


def _tc_self1(xpad, W1_self, b1):
    return pl.pallas_call(
        _self1_body, grid=(NPAD // RB,),
        in_specs=[_row_spec(D_IN), _full_spec(D_IN, D_H), _full_spec(1, D_H)],
        out_specs=_row_spec(D_H),
        out_shape=jax.ShapeDtypeStruct((NPAD, D_H), jnp.float32),
    )(xpad, W1_self, b1)


def _tc_rest1(z1, agg1, degp, W1_neigh):
    return pl.pallas_call(
        _rest1_body, grid=(NPAD // RB,),
        in_specs=[_row_spec(D_H), _chunk_spec(2), _chunk_spec(2),
                  _full_spec(D_IN, D_H)],
        out_specs=_chunk_spec(4),
        out_shape=jax.ShapeDtypeStruct((4, NPAD, 128), jnp.float32),
    )(z1, agg1, degp, W1_neigh)


def _tc_self2(h4, W2_self, b2):
    return pl.pallas_call(
        _self2_body, grid=(NPAD // RB,),
        in_specs=[_chunk_spec(4), _full_spec(D_H, D_H), _full_spec(1, D_H)],
        out_specs=_row_spec(D_H),
        out_shape=jax.ShapeDtypeStruct((NPAD, D_H), jnp.float32),
    )(h4, W2_self, b2)


RB2 = 1000


def _tc_rest2(z2, agg2, degp, W2_neigh):
    return pl.pallas_call(
        _rest2_body, grid=(N // RB2,),
        in_specs=[pl.BlockSpec((RB2, D_H), lambda i: (i, 0)),
                  pl.BlockSpec((4, RB2, 128), lambda i: (0, i, 0)),
                  pl.BlockSpec((2, RB2, 128), lambda i: (0, i, 0)),
                  _full_spec(D_H, D_H)],
        out_specs=pl.BlockSpec((RB2, D_H), lambda i: (i, 0)),
        out_shape=jax.ShapeDtypeStruct((N, D_H), jnp.float32),
    )(z2, agg2, degp, W2_neigh)


def kernel(in_feat, edge_index, W1_self, W1_neigh, b1, W2_self, W2_neigh, b2):
    src = edge_index[0].astype(jnp.int32)
    dst = edge_index[1].astype(jnp.int32)
    pad_e = EPAD - E
    srcp = jnp.concatenate([src, jnp.zeros((pad_e,), jnp.int32)])
    dstp = jnp.concatenate([dst, jnp.full((pad_e,), NPAD - 1, jnp.int32)])
    src1 = jnp.concatenate([srcp, srcp + NPAD]).reshape(2 * EPAD // B, B)
    src2 = jnp.concatenate([srcp + c * NPAD for c in range(4)]).reshape(4 * EPAD // B, B)
    dst2d = dstp.reshape(EPAD // B, B)

    xpad = jnp.pad(in_feat, ((0, NPAD - N), (0, 0)))
    x2 = xpad.reshape(NPAD, 2, 128).transpose(1, 0, 2).reshape(2 * NPAD, 128)
    zeros = jnp.zeros((B, 128), jnp.float32)
    ones = jnp.ones((B, 128), jnp.float32)

    agg1, degp = _sc_agg1(x2, src1, dst2d, zeros, ones)
    z1 = _tc_self1(xpad, W1_self, b1.reshape(1, D_H))
    h4 = _tc_rest1(z1, agg1.reshape(2, NPAD, 128), degp.reshape(2, NPAD, 128),
                   W1_neigh)
    agg2 = _sc_agg2(h4.reshape(4 * NPAD, 128), src2, dst2d, zeros)
    z2 = _tc_self2(h4, W2_self, b2.reshape(1, D_H))
    return _tc_rest2(z2, agg2.reshape(4, NPAD, 128),
                     degp.reshape(2, NPAD, 128), W2_neigh)
